# Initial kernel scaffold; baseline (speedup 1.0000x reference)
import jax
import jax.numpy as jnp
from jax.experimental import pallas as pl


def kernel(x, table, W, b):
    raise NotImplementedError("write your pallas kernel here")


# speedup vs baseline: 1.1699x; 1.1699x over previous
import functools

import jax
import jax.numpy as jnp
from jax import lax
from jax.experimental import pallas as pl
from jax.experimental.pallas import tpu as pltpu
from jax.experimental.pallas import tpu_sc as plsc

_ROWS = 8000
_IDXW = 128
_K = 8


def _table_transform(table, W, b):
    V, E = table.shape
    F = W.shape[1]
    assert V % _ROWS == 0

    def body(t_ref, w_ref, b_ref, o_ref):
        o_ref[...] = jnp.maximum(
            jnp.dot(t_ref[...], w_ref[...],
                    preferred_element_type=jnp.float32) + b_ref[...],
            0.0)

    return pl.pallas_call(
        body,
        grid=(V // _ROWS,),
        in_specs=[
            pl.BlockSpec((_ROWS, E), lambda i: (i, 0)),
            pl.BlockSpec((E, F), lambda i: (0, 0)),
            pl.BlockSpec((1, F), lambda i: (0, 0)),
        ],
        out_specs=pl.BlockSpec((_ROWS, F), lambda i: (i, 0)),
        out_shape=jax.ShapeDtypeStruct((V, F), jnp.float32),
    )(table, W, b.reshape(1, F))


@functools.lru_cache(maxsize=None)
def _make_gather(F, n_rows):
    info = plsc.get_sparse_core_info()
    NC, NS = info.num_cores, info.num_subcores
    NW = NC * NS
    rows_per_w = n_rows // NW
    steps = rows_per_w // _K
    assert steps * _K == rows_per_w
    mesh = plsc.VectorSubcoreMesh(core_axis_name="c", subcore_axis_name="s")

    @functools.partial(
        pl.kernel, mesh=mesh,
        compiler_params=pltpu.CompilerParams(use_tc_tiling_on_sc=False),
        out_type=jax.ShapeDtypeStruct((n_rows, _IDXW, F), jnp.float32),
        scratch_types=[
            pltpu.VMEM((_K, _IDXW), jnp.int32),
            pltpu.VMEM((_K, _IDXW, F), jnp.float32),
            pltpu.SemaphoreType.DMA,
        ],
    )
    def k(t3_hbm, idx_hbm, out_hbm, idx_v, rows_v, sem):
        wid = lax.axis_index("s") * NC + lax.axis_index("c")
        base = wid * rows_per_w

        def body(s, carry):
            off = base + s * _K
            pltpu.sync_copy(idx_hbm.at[pl.ds(off, _K)], idx_v)
            cps = [pltpu.async_copy(t3_hbm.at[idx_v.at[j]], rows_v.at[j], sem)
                   for j in range(_K)]
            for cp in cps:
                cp.wait()
            pltpu.sync_copy(rows_v, out_hbm.at[pl.ds(off, _K)])
            return carry

        lax.fori_loop(0, steps, body, 0)

    return k


def kernel(x, table, W, b):
    V, E = table.shape
    F = W.shape[1]
    Bsz, L = x.shape
    N = Bsz * L
    t3 = _table_transform(table, W, b)
    idx = x.reshape(N // _IDXW, _IDXW)
    out = _make_gather(F, N // _IDXW)(t3, idx)
    return out.reshape(Bsz, L, F)


# speedup vs baseline: 1.9185x; 1.6398x over previous
import functools

import jax
import jax.numpy as jnp
from jax import lax
from jax.experimental import pallas as pl
from jax.experimental.pallas import tpu as pltpu
from jax.experimental.pallas import tpu_sc as plsc

_ROWS = 8192
_BR = 16


def _table_transform(table_t, W, b):
    E, V = table_t.shape
    F = W.shape[1]

    def body(t_ref, w_ref, b_ref, o_ref):
        acc = lax.dot_general(
            t_ref[...], w_ref[...],
            dimension_numbers=(((0,), (0,)), ((), ())),
            preferred_element_type=jnp.float32)
        o_ref[...] = jnp.maximum(acc + b_ref[...], 0.0)

    return pl.pallas_call(
        body,
        grid=(pl.cdiv(V, _ROWS),),
        in_specs=[
            pl.BlockSpec((E, _ROWS), lambda i: (0, i)),
            pl.BlockSpec((E, F), lambda i: (0, 0)),
            pl.BlockSpec((1, F), lambda i: (0, 0)),
        ],
        out_specs=pl.BlockSpec((_ROWS, F), lambda i: (i, 0)),
        out_shape=jax.ShapeDtypeStruct((V, F), jnp.float32),
    )(table_t, W, b.reshape(1, F))


@functools.lru_cache(maxsize=None)
def _make_gather(F, B, L):
    info = plsc.get_sparse_core_info()
    NC, NS = info.num_cores, info.num_subcores
    NW = NC * NS
    rows_per_w = B // NW
    steps = rows_per_w // _BR
    assert steps * _BR == rows_per_w
    mesh = plsc.VectorSubcoreMesh(core_axis_name="c", subcore_axis_name="s")

    @functools.partial(
        pl.kernel, mesh=mesh,
        compiler_params=pltpu.CompilerParams(use_tc_tiling_on_sc=False),
        out_type=jax.ShapeDtypeStruct((B, L, F), jnp.float32),
        scratch_types=[
            pltpu.VMEM((_BR, L), jnp.int32),
            pltpu.VMEM((_BR, L, F), jnp.float32),
            pltpu.SemaphoreType.DMA,
        ],
    )
    def k(t3_hbm, x_hbm, out_hbm, idx_v, rows_v, sem):
        wid = lax.axis_index("s") * NC + lax.axis_index("c")
        base = wid * rows_per_w

        def body(s, carry):
            r0 = base + s * _BR
            pltpu.sync_copy(x_hbm.at[pl.ds(r0, _BR)], idx_v)
            cps = [pltpu.async_copy(t3_hbm.at[idx_v.at[j]], rows_v.at[j], sem)
                   for j in range(_BR)]
            for cp in cps:
                cp.wait()
            pltpu.sync_copy(rows_v, out_hbm.at[pl.ds(r0, _BR)])
            return carry

        lax.fori_loop(0, steps, body, 0)

    return k


def kernel(x, table, W, b):
    V, E = table.shape
    F = W.shape[1]
    B, L = x.shape
    t3 = _table_transform(table.T, W, b)
    return _make_gather(F, B, L)(t3, x)


# speedup vs baseline: 2.6840x; 1.3990x over previous
import functools

import jax
import jax.numpy as jnp
from jax import lax
from jax.experimental import pallas as pl
from jax.experimental.pallas import tpu as pltpu
from jax.experimental.pallas import tpu_sc as plsc

_C4 = 2048
_BR = 16


def _table_transform(table_t, W, b):
    E, V = table_t.shape
    F = W.shape[1]

    def body(t0, t1, t2, t3_, w_ref, b_ref, o_ref):
        outs = []
        for t_ref in (t0, t1, t2, t3_):
            acc = lax.dot_general(
                t_ref[...], w_ref[...],
                dimension_numbers=(((0,), (0,)), ((), ())),
                preferred_element_type=jnp.float32)
            outs.append(jnp.maximum(acc + b_ref[...], 0.0))
        o_ref[...] = jnp.concatenate(outs, axis=1)

    last_blk = (V - 1) // _C4
    t_spec = [
        pl.BlockSpec((E, _C4),
                     lambda i, a=a: (0, jnp.minimum(4 * i + a, last_blk)))
        for a in range(4)
    ]
    grid = pl.cdiv(V, 4 * _C4)
    return pl.pallas_call(
        body,
        grid=(grid,),
        in_specs=t_spec + [
            pl.BlockSpec((E, F), lambda i: (0, 0)),
            pl.BlockSpec((1, F), lambda i: (0, 0)),
        ],
        out_specs=pl.BlockSpec((_C4, 4 * F), lambda i: (i, 0)),
        out_shape=jax.ShapeDtypeStruct((grid * _C4, 4 * F), jnp.float32),
    )(table_t, table_t, table_t, table_t, W, b.reshape(1, F))


@functools.lru_cache(maxsize=None)
def _make_gather(F, B, L):
    info = plsc.get_sparse_core_info()
    NC, NS = info.num_cores, info.num_subcores
    NW = NC * NS
    rows_per_w = B // NW
    steps = rows_per_w // _BR
    assert steps * _BR == rows_per_w
    mesh = plsc.VectorSubcoreMesh(core_axis_name="c", subcore_axis_name="s")

    @functools.partial(
        pl.kernel, mesh=mesh,
        compiler_params=pltpu.CompilerParams(use_tc_tiling_on_sc=False),
        out_type=jax.ShapeDtypeStruct((B, L, F), jnp.float32),
        scratch_types=[
            pltpu.VMEM((_BR, L), jnp.int32),
            pltpu.VMEM((_BR, L, F), jnp.float32),
            pltpu.SemaphoreType.DMA,
        ],
    )
    def k(t3_hbm, x_hbm, out_hbm, idx_v, rows_v, sem):
        wid = lax.axis_index("s") * NC + lax.axis_index("c")
        base = wid * rows_per_w

        def body(s, carry):
            r0 = base + s * _BR
            pltpu.sync_copy(x_hbm.at[pl.ds(r0, _BR)], idx_v)
            cps = [pltpu.async_copy(t3_hbm.at[idx_v.at[j]], rows_v.at[j], sem)
                   for j in range(_BR)]
            for cp in cps:
                cp.wait()
            pltpu.sync_copy(rows_v, out_hbm.at[pl.ds(r0, _BR)])
            return carry

        lax.fori_loop(0, steps, body, 0)

    return k


def kernel(x, table, W, b):
    V, E = table.shape
    F = W.shape[1]
    B, L = x.shape
    t3p = _table_transform(table.T, W, b)
    t3 = t3p.reshape(t3p.shape[0] * 4, F)
    x2 = ((x >> 13) << 13) | ((x & 2047) << 2) | ((x >> 11) & 3)
    return _make_gather(F, B, L)(t3, x2)


# speedup vs baseline: 2.9823x; 1.1112x over previous
import functools

import jax
import jax.numpy as jnp
from jax import lax
from jax.experimental import pallas as pl
from jax.experimental.pallas import tpu as pltpu
from jax.experimental.pallas import tpu_sc as plsc

_C4 = 2048
_BR = 16


def _table_transform(table_t, W, b):
    E, V = table_t.shape
    F = W.shape[1]

    def body(t0, t1, t2, t3_, w_ref, b_ref, o_ref):
        stack = jnp.concatenate(
            [t0[...], t1[...], t2[...], t3_[...]], axis=0)
        acc = lax.dot_general(
            stack, w_ref[...],
            dimension_numbers=(((0,), (0,)), ((), ())),
            preferred_element_type=jnp.float32)
        o_ref[...] = jnp.maximum(acc + b_ref[...], 0.0)

    last_blk = (V - 1) // _C4
    t_spec = [
        pl.BlockSpec((E, _C4),
                     lambda i, a=a: (0, jnp.minimum(4 * i + a, last_blk)))
        for a in range(4)
    ]
    grid = pl.cdiv(V, 4 * _C4)
    return pl.pallas_call(
        body,
        grid=(grid,),
        in_specs=t_spec + [
            pl.BlockSpec((4 * E, 4 * F), lambda i: (0, 0)),
            pl.BlockSpec((1, 4 * F), lambda i: (0, 0)),
        ],
        out_specs=pl.BlockSpec((_C4, 4 * F), lambda i: (i, 0)),
        out_shape=jax.ShapeDtypeStruct((grid * _C4, 4 * F), jnp.float32),
    )(table_t, table_t, table_t, table_t,
      jax.scipy.linalg.block_diag(W, W, W, W),
      jnp.tile(b, 4).reshape(1, 4 * F))


@functools.lru_cache(maxsize=None)
def _make_gather(F, B, L):
    info = plsc.get_sparse_core_info()
    NC, NS = info.num_cores, info.num_subcores
    NW = NC * NS
    rows_per_w = B // NW
    steps = rows_per_w // _BR
    assert steps * _BR == rows_per_w
    mesh = plsc.VectorSubcoreMesh(core_axis_name="c", subcore_axis_name="s")

    @functools.partial(
        pl.kernel, mesh=mesh,
        compiler_params=pltpu.CompilerParams(use_tc_tiling_on_sc=False),
        out_type=jax.ShapeDtypeStruct((B, L, F), jnp.float32),
        scratch_types=[
            pltpu.VMEM((_BR, L), jnp.int32),
            pltpu.VMEM((_BR, L, F), jnp.float32),
            pltpu.SemaphoreType.DMA,
        ],
    )
    def k(t3_hbm, x_hbm, out_hbm, idx_v, rows_v, sem):
        wid = lax.axis_index("s") * NC + lax.axis_index("c")
        base = wid * rows_per_w

        def body(s, carry):
            r0 = base + s * _BR
            pltpu.sync_copy(x_hbm.at[pl.ds(r0, _BR)], idx_v)
            cps = [pltpu.async_copy(t3_hbm.at[idx_v.at[j]], rows_v.at[j], sem)
                   for j in range(_BR)]
            for cp in cps:
                cp.wait()
            pltpu.sync_copy(rows_v, out_hbm.at[pl.ds(r0, _BR)])
            return carry

        lax.fori_loop(0, steps, body, 0)

    return k


def kernel(x, table, W, b):
    V, E = table.shape
    F = W.shape[1]
    B, L = x.shape
    t3p = _table_transform(table.T, W, b)
    t3 = t3p.reshape(t3p.shape[0] * 4, F)
    x2 = ((x >> 13) << 13) | ((x & 2047) << 2) | ((x >> 11) & 3)
    return _make_gather(F, B, L)(t3, x2)


# speedup vs baseline: 3.0802x; 1.0328x over previous
import functools

import jax
import jax.numpy as jnp
from jax import lax
from jax.experimental import pallas as pl
from jax.experimental.pallas import tpu as pltpu
from jax.experimental.pallas import tpu_sc as plsc

_C4 = 2048
_BR = 16


def _table_transform(table_t, W, b):
    E, V = table_t.shape
    F = W.shape[1]

    def body(t0, t1, t2, t3_, w_ref, b_ref, o_ref):
        stack = jnp.concatenate(
            [t0[...], t1[...], t2[...], t3_[...]], axis=0)
        acc = lax.dot_general(
            stack, w_ref[...],
            dimension_numbers=(((0,), (0,)), ((), ())),
            preferred_element_type=jnp.float32)
        o_ref[...] = jnp.maximum(acc + b_ref[...], 0.0)

    last_blk = (V - 1) // _C4
    t_spec = [
        pl.BlockSpec((E, _C4),
                     lambda i, a=a: (0, jnp.minimum(4 * i + a, last_blk)))
        for a in range(4)
    ]
    grid = pl.cdiv(V, 4 * _C4)
    return pl.pallas_call(
        body,
        grid=(grid,),
        in_specs=t_spec + [
            pl.BlockSpec((4 * E, 4 * F), lambda i: (0, 0)),
            pl.BlockSpec((1, 4 * F), lambda i: (0, 0)),
        ],
        out_specs=pl.BlockSpec((_C4, 4 * F), lambda i: (i, 0)),
        out_shape=jax.ShapeDtypeStruct((grid * _C4, 4 * F), jnp.float32),
    )(table_t, table_t, table_t, table_t,
      jax.scipy.linalg.block_diag(W, W, W, W),
      jnp.tile(b, 4).reshape(1, 4 * F))


@functools.lru_cache(maxsize=None)
def _make_gather(F, B, L):
    info = plsc.get_sparse_core_info()
    NC, NS = info.num_cores, info.num_subcores
    NW = NC * NS
    rows_per_w = B // NW
    steps = rows_per_w // _BR
    assert steps * _BR == rows_per_w
    mesh = plsc.VectorSubcoreMesh(core_axis_name="c", subcore_axis_name="s")

    assert steps % 2 == 0 and steps >= 4

    @functools.partial(
        pl.kernel, mesh=mesh,
        compiler_params=pltpu.CompilerParams(use_tc_tiling_on_sc=False),
        out_type=jax.ShapeDtypeStruct((B, L, F), jnp.float32),
        scratch_types=[
            pltpu.VMEM((2, _BR, L), jnp.int32),
            pltpu.VMEM((2, _BR, L, F), jnp.float32),
            pltpu.SemaphoreType.DMA,
            pltpu.SemaphoreType.DMA,
        ],
    )
    def k(t3_hbm, x_hbm, out_hbm, idx_v, rows_v, sem0, sem1):
        wid = lax.axis_index("s") * NC + lax.axis_index("c")
        base = wid * rows_per_w
        sems = (sem0, sem1)

        def fire(s, slot):
            pltpu.sync_copy(x_hbm.at[pl.ds(base + s * _BR, _BR)],
                            idx_v.at[slot])
            return [pltpu.async_copy(t3_hbm.at[idx_v.at[slot].at[j]],
                                     rows_v.at[slot].at[j], sems[slot])
                    for j in range(_BR)]

        def drain_write(s, slot, cps):
            for cp in cps:
                cp.wait()
            pltpu.sync_copy(rows_v.at[slot],
                            out_hbm.at[pl.ds(base + s * _BR, _BR)])

        cps0 = fire(0, 0)

        def body(g, carry):
            s0 = 2 * g
            cps1 = fire(s0 + 1, 1)
            drain_write(s0, 0, cps0)
            cps0b = fire(s0 + 2, 0)
            drain_write(s0 + 1, 1, cps1)
            del cps0b
            return carry

        lax.fori_loop(0, steps // 2 - 1, body, 0)
        cps1 = fire(steps - 1, 1)
        drain_write(steps - 2, 0, cps0)
        drain_write(steps - 1, 1, cps1)

    return k


def kernel(x, table, W, b):
    V, E = table.shape
    F = W.shape[1]
    B, L = x.shape
    t3p = _table_transform(table.T, W, b)
    t3 = t3p.reshape(t3p.shape[0] * 4, F)
    x2 = ((x >> 13) << 13) | ((x & 2047) << 2) | ((x >> 11) & 3)
    return _make_gather(F, B, L)(t3, x2)


# speedup vs baseline: 3.2483x; 1.0546x over previous
import functools

import jax
import jax.numpy as jnp
from jax import lax
from jax.experimental import pallas as pl
from jax.experimental.pallas import tpu as pltpu
from jax.experimental.pallas import tpu_sc as plsc

_C4 = 4096
_SH = _C4.bit_length() - 1
_BR = 16


def _table_transform(table_t, W, b):
    E, V = table_t.shape
    F = W.shape[1]

    def body(t0, t1, t2, t3_, w_ref, b_ref, o_ref):
        stack = jnp.concatenate(
            [t0[...], t1[...], t2[...], t3_[...]], axis=0)
        acc = lax.dot_general(
            stack, w_ref[...],
            dimension_numbers=(((0,), (0,)), ((), ())),
            preferred_element_type=jnp.float32)
        o_ref[...] = jnp.maximum(acc + b_ref[...], 0.0)

    last_blk = (V - 1) // _C4
    t_spec = [
        pl.BlockSpec((E, _C4),
                     lambda i, a=a: (0, jnp.minimum(4 * i + a, last_blk)))
        for a in range(4)
    ]
    grid = pl.cdiv(V, 4 * _C4)
    return pl.pallas_call(
        body,
        grid=(grid,),
        in_specs=t_spec + [
            pl.BlockSpec((4 * E, 4 * F), lambda i: (0, 0)),
            pl.BlockSpec((1, 4 * F), lambda i: (0, 0)),
        ],
        out_specs=pl.BlockSpec((_C4, 4 * F), lambda i: (i, 0)),
        out_shape=jax.ShapeDtypeStruct((grid * _C4, 4 * F), jnp.float32),
    )(table_t, table_t, table_t, table_t,
      jax.scipy.linalg.block_diag(W, W, W, W),
      jnp.tile(b, 4).reshape(1, 4 * F))


@functools.lru_cache(maxsize=None)
def _make_gather(F, B, L):
    info = plsc.get_sparse_core_info()
    NC, NS = info.num_cores, info.num_subcores
    NW = NC * NS
    rows_per_w = B // NW
    steps = rows_per_w // _BR
    assert steps * _BR == rows_per_w
    mesh = plsc.VectorSubcoreMesh(core_axis_name="c", subcore_axis_name="s")

    assert steps % 2 == 0 and steps >= 4

    @functools.partial(
        pl.kernel, mesh=mesh,
        compiler_params=pltpu.CompilerParams(use_tc_tiling_on_sc=False),
        out_type=jax.ShapeDtypeStruct((B, L, F), jnp.float32),
        scratch_types=[
            pltpu.VMEM((2, _BR, L), jnp.int32),
            pltpu.VMEM((2, _BR, L, F), jnp.float32),
            pltpu.SemaphoreType.DMA,
            pltpu.SemaphoreType.DMA,
        ],
    )
    def k(t3_hbm, x_hbm, out_hbm, idx_v, rows_v, sem0, sem1):
        wid = lax.axis_index("s") * NC + lax.axis_index("c")
        base = wid * rows_per_w
        sems = (sem0, sem1)

        def fire(s, slot):
            pltpu.sync_copy(x_hbm.at[pl.ds(base + s * _BR, _BR)],
                            idx_v.at[slot])
            return [pltpu.async_copy(t3_hbm.at[idx_v.at[slot].at[j]],
                                     rows_v.at[slot].at[j], sems[slot])
                    for j in range(_BR)]

        def drain_write(s, slot, cps):
            for cp in cps:
                cp.wait()
            pltpu.sync_copy(rows_v.at[slot],
                            out_hbm.at[pl.ds(base + s * _BR, _BR)])

        cps0 = fire(0, 0)

        def body(g, carry):
            s0 = 2 * g
            cps1 = fire(s0 + 1, 1)
            drain_write(s0, 0, cps0)
            cps0b = fire(s0 + 2, 0)
            drain_write(s0 + 1, 1, cps1)
            del cps0b
            return carry

        lax.fori_loop(0, steps // 2 - 1, body, 0)
        cps1 = fire(steps - 1, 1)
        drain_write(steps - 2, 0, cps0)
        drain_write(steps - 1, 1, cps1)

    return k


def kernel(x, table, W, b):
    V, E = table.shape
    F = W.shape[1]
    B, L = x.shape
    t3p = _table_transform(table.T, W, b)
    t3 = t3p.reshape(t3p.shape[0] * 4, F)
    x2 = (((x >> (_SH + 2)) << (_SH + 2))
          | ((x & (_C4 - 1)) << 2) | ((x >> _SH) & 3))
    return _make_gather(F, B, L)(t3, x2)
